# baseline (speedup 1.0000x reference)
import functools

import jax
import jax.numpy as jnp
from jax import lax
from jax.experimental import pallas as pl
from jax.experimental.pallas import tpu as pltpu
from jax.experimental.pallas import tpu_sc as plsc

_N = 10000
_F = 128
_FH = _F // 2
_E = 320000
_NC = 2
_NS = 16
_NW = _NC * _NS
_CHUNK = 128
_PER_W = _E // _NW
_NCHUNK = 80
_PER_W_PAD = _NCHUNK * _CHUNK
_NPAD = 10240
_ROWS_PER_TILE = _NPAD // _NS
_ZROWS = 128


def _sc_body(xl_hbm, xr_hbm, src_hbm, dst_hbm, ew_hbm, outl_hbm, outr_hbm,
             src_v, dst_v, ew_v, rows_v, rows_v1, zrow_v, acc_sh, sem, sem1):
    c = lax.axis_index("c")
    s = lax.axis_index("s")
    wid = c * _NS + s
    base = s * _ROWS_PER_TILE
    zeros16 = jnp.zeros((16,), jnp.float32)

    def _zero_row(i, carry):
        for f in range(_FH // 16):
            zrow_v[i, pl.ds(f * 16, 16)] = zeros16
        return carry

    def _zero_acc_slice():
        for k in range(_ROWS_PER_TILE // _ZROWS):
            pltpu.sync_copy(zrow_v, acc_sh.at[pl.ds(base + k * _ZROWS, _ZROWS)])

    lax.fori_loop(0, _ZROWS, _zero_row, 0)
    _zero_acc_slice()

    pltpu.sync_copy(src_hbm.at[wid], src_v)
    pltpu.sync_copy(dst_hbm.at[wid], dst_v)
    pltpu.sync_copy(ew_hbm.at[wid], ew_v)

    plsc.subcore_barrier()

    def _accumulate(x_hbm):
        buf = (rows_v, rows_v1)
        sems = (sem, sem1)
        pltpu.async_copy(x_hbm.at[src_v.at[0]], buf[0], sems[0])

        def _chunk2(i, carry):
            for b in range(2):
                j = 2 * i + b
                pltpu.make_async_copy(x_hbm.at[src_v.at[j]], buf[b],
                                      sems[b]).wait()

                @pl.when(j < _NCHUNK - 1)
                def _():
                    pltpu.async_copy(x_hbm.at[src_v.at[j + 1]], buf[1 - b],
                                     sems[1 - b])

                for blk in range(_CHUNK // 16):
                    wv16 = ew_v[j, pl.ds(blk * 16, 16)]
                    for e in range(16):
                        wv = jnp.full((16,), wv16[e], jnp.float32)
                        row = blk * 16 + e
                        for f in range(_FH // 16):
                            sl = pl.ds(f * 16, 16)
                            buf[b][row, sl] = buf[b][row, sl] * wv
                pltpu.sync_copy(buf[b], acc_sh.at[dst_v.at[j]], add=True)
            return carry

        lax.fori_loop(0, _NCHUNK // 2, _chunk2, 0)

    tile_rows = pl.ds(base, _ROWS_PER_TILE)

    _accumulate(xl_hbm)
    plsc.subcore_barrier()
    pltpu.sync_copy(acc_sh.at[tile_rows], outl_hbm.at[c, tile_rows])
    _zero_acc_slice()
    plsc.subcore_barrier()

    _accumulate(xr_hbm)
    plsc.subcore_barrier()
    pltpu.sync_copy(acc_sh.at[tile_rows], outr_hbm.at[c, tile_rows])


@jax.jit
def _sc_segsum(xl, xr, src3, dst3, ew3):
    fn = pl.kernel(
        _sc_body,
        out_type=(jax.ShapeDtypeStruct((_NC, _NPAD, _FH), jnp.float32),
                  jax.ShapeDtypeStruct((_NC, _NPAD, _FH), jnp.float32)),
        mesh=plsc.VectorSubcoreMesh(core_axis_name="c", subcore_axis_name="s"),
        compiler_params=pltpu.CompilerParams(use_tc_tiling_on_sc=False),
        scratch_types=[
            pltpu.VMEM((_NCHUNK, _CHUNK), jnp.int32),
            pltpu.VMEM((_NCHUNK, _CHUNK), jnp.int32),
            pltpu.VMEM((_NCHUNK, _CHUNK), jnp.float32),
            pltpu.VMEM((_CHUNK, _FH), jnp.float32),
            pltpu.VMEM((_CHUNK, _FH), jnp.float32),
            pltpu.VMEM((_ZROWS, _FH), jnp.float32),
            pltpu.VMEM_SHARED((_NPAD, _FH), jnp.float32),
            pltpu.SemaphoreType.DMA,
            pltpu.SemaphoreType.DMA,
        ],
    )
    return fn(xl, xr, src3, dst3, ew3)


_BN = 1000


def _cell_body(with_head, p0l_ref, p1l_ref, p0r_ref, p1r_ref, hs_ref, deg_ref,
               wcat_ref, bcat_ref, ltop_ref, lbzr_ref, lbh_ref, lbias_ref,
               linw_ref, linb_ref, h_ref, y_ref, out_ref):
    prec = lax.Precision.DEFAULT
    invd = 1.0 / jnp.clip(deg_ref[...], 1.0, None)
    xsl = (p0l_ref[...] + p1l_ref[...]) * invd
    xsr = (p0r_ref[...] + p1r_ref[...]) * invd
    a = (jnp.dot(xsl, wcat_ref[0:_FH, :], precision=prec,
                 preferred_element_type=jnp.float32)
         + jnp.dot(xsr, wcat_ref[_FH:_F, :], precision=prec,
                   preferred_element_type=jnp.float32)
         + bcat_ref[...])
    hs = hs_ref[...]
    pzr_b = jnp.dot(hs, lbzr_ref[...], precision=prec,
                    preferred_element_type=jnp.float32)
    lb = lbias_ref[...]
    pz = jnp.dot(a[:, 0:_F], ltop_ref[..., 0:_F], precision=prec,
                 preferred_element_type=jnp.float32)
    pr = jnp.dot(a[:, _F:2 * _F], ltop_ref[..., _F:2 * _F], precision=prec,
                 preferred_element_type=jnp.float32)
    ph = jnp.dot(a[:, 2 * _F:3 * _F], ltop_ref[..., 2 * _F:3 * _F],
                 precision=prec, preferred_element_type=jnp.float32)
    z = jax.nn.sigmoid(pz + pzr_b[:, 0:_F] + lb[:, 0:_F])
    r = jax.nn.sigmoid(pr + pzr_b[:, _F:2 * _F] + lb[:, _F:2 * _F])
    ph_b = jnp.dot(hs * r, lbh_ref[...], precision=prec,
                   preferred_element_type=jnp.float32)
    htil = jnp.tanh(ph + ph_b + lb[:, 2 * _F:3 * _F])
    hn = z * hs + (1.0 - z) * htil
    h_ref[...] = hn
    y = jnp.maximum(hn, 0.0)
    y_ref[...] = y
    if with_head:
        out_ref[...] = jnp.dot(y, linw_ref[...], precision=prec,
                               preferred_element_type=jnp.float32) + linb_ref[...]
    else:
        out_ref[...] = jnp.zeros(out_ref.shape, out_ref.dtype)


def _cell_call(partsl, partsr, hs, degf, wcat, bcat, ltop, lbzr, lbh, lbias,
               linw, linb, with_head):
    grid = _N // _BN
    half_spec = pl.BlockSpec((_BN, _FH), lambda i: (i, 0))
    row_spec = pl.BlockSpec((_BN, _F), lambda i: (i, 0))
    deg_spec = pl.BlockSpec((_BN, 1), lambda i: (i, 0))
    full = lambda *shape: pl.BlockSpec(shape, lambda i: tuple(0 for _ in shape))
    out_shapes = [
        jax.ShapeDtypeStruct((_N, _F), jnp.float32),
        jax.ShapeDtypeStruct((_N, _F), jnp.float32),
        jax.ShapeDtypeStruct((_N, 1), jnp.float32),
    ]
    out_specs = [row_spec, row_spec, pl.BlockSpec((_BN, 1), lambda i: (i, 0))]
    h, y, out = pl.pallas_call(
        functools.partial(_cell_body, with_head),
        grid=(grid,),
        in_specs=[half_spec, half_spec, half_spec, half_spec, row_spec,
                  deg_spec,
                  full(_F, 3 * _F), full(1, 3 * _F), full(_F, 3 * _F),
                  full(_F, 2 * _F), full(_F, _F), full(1, 3 * _F),
                  full(_F, 1), full(1, 1)],
        out_specs=out_specs,
        out_shape=out_shapes,
    )(partsl[0], partsl[1], partsr[0], partsr[1], hs, degf, wcat, bcat, ltop,
      lbzr, lbh, lbias, linw, linb)
    return h, y, out


def _edge_layout(idx_or_w, pad_value):
    a = idx_or_w.reshape(_NW, _PER_W)
    a = jnp.pad(a, ((0, 0), (0, _PER_W_PAD - _PER_W)), constant_values=pad_value)
    return a.reshape(_NW, _NCHUNK, _CHUNK)


def _pack_cell_weights(Wz, bz, Lzw, Lzb, Wr, br, Lrw, Lrb, Wh, bh, Lhw, Lhb):
    wcat = jnp.concatenate([Wz, Wr, Wh], axis=1)
    bcat = jnp.concatenate([bz, br, bh]).reshape(1, 3 * _F)
    ltop = jnp.concatenate([Lzw[:_F], Lrw[:_F], Lhw[:_F]], axis=1)
    lbzr = jnp.concatenate([Lzw[_F:], Lrw[_F:]], axis=1)
    lbh = Lhw[_F:]
    lbias = jnp.concatenate([Lzb, Lrb, Lhb]).reshape(1, 3 * _F)
    return wcat, bcat, ltop, lbzr, lbh, lbias


def kernel(x, edge0, edge1, edge_weight0, edge_weight1, prev_h0, prev_h1,
           deg0, deg1, deg2, time, status,
           Wz1, bz1, Lz1w, Lz1b, Wr1, br1, Lr1w, Lr1b, Wh1, bh1, Lh1w, Lh1b,
           Wz2, bz2, Lz2w, Lz2b, Wr2, br2, Lr2w, Lr2b, Wh2, bh2, Lh2w, Lh2b,
           lin_w, lin_b):
    del deg0, time, status
    src1 = _edge_layout(edge1[0].astype(jnp.int32), 0)
    dst1 = _edge_layout(edge1[1].astype(jnp.int32), 0)
    ew1 = _edge_layout(edge_weight1, 0.0)
    src0 = _edge_layout(edge0[0].astype(jnp.int32), 0)
    dst0 = _edge_layout(edge0[1].astype(jnp.int32), 0)
    ew0 = _edge_layout(edge_weight0, 0.0)

    w1 = _pack_cell_weights(Wz1, bz1, Lz1w, Lz1b, Wr1, br1, Lr1w, Lr1b,
                            Wh1, bh1, Lh1w, Lh1b)
    w2 = _pack_cell_weights(Wz2, bz2, Lz2w, Lz2b, Wr2, br2, Lr2w, Lr2b,
                            Wh2, bh2, Lh2w, Lh2b)
    deg2f = deg2.astype(jnp.float32).reshape(_N, 1)
    deg1f = deg1.astype(jnp.float32).reshape(_N, 1)
    linw = lin_w.reshape(_F, 1)
    linb = lin_b.reshape(1, 1)

    xl = x[:, :_FH]
    xr = x[:, _FH:]
    p1l, p1r = _sc_segsum(xl, xr, src1, dst1, ew1)
    h1, y1, _ = _cell_call(p1l, p1r, prev_h0, deg2f, *w1, linw, linb,
                           with_head=False)
    p2l, p2r = _sc_segsum(y1[:, :_FH], y1[:, _FH:], src0, dst0, ew0)
    h2, _, out = _cell_call(p2l, p2r, prev_h1, deg1f, *w2, linw, linb,
                            with_head=True)
    return (out, h1, h2)


# speedup vs baseline: 1.1131x; 1.1131x over previous
import functools

import jax
import jax.numpy as jnp
from jax import lax
from jax.experimental import pallas as pl
from jax.experimental.pallas import tpu as pltpu
from jax.experimental.pallas import tpu_sc as plsc

_N = 10000
_F = 128
_FH = _F // 2
_E = 320000
_NC = 2
_NS = 16
_NW = _NC * _NS
_CHUNK = 128
_PER_W = _E // _NW
_NCHUNK = -(-_PER_W // _CHUNK)
_PER_W_PAD = _NCHUNK * _CHUNK
_NPAD = 10240
_ROWS_PER_TILE = _NPAD // _NS
_ZROWS = 128


def _sc_body(xl_hbm, xr_hbm, src_hbm, dst_hbm, ew_hbm, outl_hbm, outr_hbm,
             src_v, dst_v, ew_v, rows_v, zrow_v, acc_sh, sem):
    c = lax.axis_index("c")
    s = lax.axis_index("s")
    wid = c * _NS + s
    base = s * _ROWS_PER_TILE
    zeros16 = jnp.zeros((16,), jnp.float32)

    def _zero_row(i, carry):
        for f in range(_FH // 16):
            zrow_v[i, pl.ds(f * 16, 16)] = zeros16
        return carry

    def _zero_acc_slice():
        for k in range(_ROWS_PER_TILE // _ZROWS):
            pltpu.sync_copy(zrow_v, acc_sh.at[pl.ds(base + k * _ZROWS, _ZROWS)])

    lax.fori_loop(0, _ZROWS, _zero_row, 0)
    _zero_acc_slice()

    pltpu.sync_copy(src_hbm.at[wid], src_v)
    pltpu.sync_copy(dst_hbm.at[wid], dst_v)
    pltpu.sync_copy(ew_hbm.at[wid], ew_v)

    plsc.subcore_barrier()

    def _accumulate(x_hbm):
        def _chunk(j, carry):
            pltpu.async_copy(x_hbm.at[src_v.at[j]], rows_v, sem).wait()
            for b in range(_CHUNK // 16):
                wv16 = ew_v[j, pl.ds(b * 16, 16)]
                for e in range(16):
                    wv = jnp.full((16,), wv16[e], jnp.float32)
                    row = b * 16 + e
                    for f in range(_FH // 16):
                        sl = pl.ds(f * 16, 16)
                        rows_v[row, sl] = rows_v[row, sl] * wv
            pltpu.sync_copy(rows_v, acc_sh.at[dst_v.at[j]], add=True)
            return carry

        lax.fori_loop(0, _NCHUNK, _chunk, 0)

    tile_rows = pl.ds(base, _ROWS_PER_TILE)

    _accumulate(xl_hbm)
    plsc.subcore_barrier()
    pltpu.sync_copy(acc_sh.at[tile_rows], outl_hbm.at[c, tile_rows])
    _zero_acc_slice()
    plsc.subcore_barrier()

    _accumulate(xr_hbm)
    plsc.subcore_barrier()
    pltpu.sync_copy(acc_sh.at[tile_rows], outr_hbm.at[c, tile_rows])


@jax.jit
def _sc_segsum(xl, xr, src3, dst3, ew3):
    fn = pl.kernel(
        _sc_body,
        out_type=(jax.ShapeDtypeStruct((_NC, _NPAD, _FH), jnp.float32),
                  jax.ShapeDtypeStruct((_NC, _NPAD, _FH), jnp.float32)),
        mesh=plsc.VectorSubcoreMesh(core_axis_name="c", subcore_axis_name="s"),
        compiler_params=pltpu.CompilerParams(use_tc_tiling_on_sc=False),
        scratch_types=[
            pltpu.VMEM((_NCHUNK, _CHUNK), jnp.int32),
            pltpu.VMEM((_NCHUNK, _CHUNK), jnp.int32),
            pltpu.VMEM((_NCHUNK, _CHUNK), jnp.float32),
            pltpu.VMEM((_CHUNK, _FH), jnp.float32),
            pltpu.VMEM((_ZROWS, _FH), jnp.float32),
            pltpu.VMEM_SHARED((_NPAD, _FH), jnp.float32),
            pltpu.SemaphoreType.DMA,
        ],
    )
    return fn(xl, xr, src3, dst3, ew3)


_BN = 2000


def _cell_body(with_head, p0l_ref, p1l_ref, p0r_ref, p1r_ref, hs_ref, deg_ref,
               wcat_ref, bcat_ref, ltop_ref, lbzr_ref, lbh_ref, lbias_ref,
               linw_ref, linb_ref, h_ref, y_ref, out_ref):
    prec = lax.Precision.DEFAULT
    invd = 1.0 / jnp.clip(deg_ref[...], 1.0, None)
    xsl = (p0l_ref[...] + p1l_ref[...]) * invd
    xsr = (p0r_ref[...] + p1r_ref[...]) * invd
    a = (jnp.dot(xsl, wcat_ref[0:_FH, :], precision=prec,
                 preferred_element_type=jnp.float32)
         + jnp.dot(xsr, wcat_ref[_FH:_F, :], precision=prec,
                   preferred_element_type=jnp.float32)
         + bcat_ref[...])
    hs = hs_ref[...]
    pzr_b = jnp.dot(hs, lbzr_ref[...], precision=prec,
                    preferred_element_type=jnp.float32)
    lb = lbias_ref[...]
    pz = jnp.dot(a[:, 0:_F], ltop_ref[..., 0:_F], precision=prec,
                 preferred_element_type=jnp.float32)
    pr = jnp.dot(a[:, _F:2 * _F], ltop_ref[..., _F:2 * _F], precision=prec,
                 preferred_element_type=jnp.float32)
    ph = jnp.dot(a[:, 2 * _F:3 * _F], ltop_ref[..., 2 * _F:3 * _F],
                 precision=prec, preferred_element_type=jnp.float32)
    z = jax.nn.sigmoid(pz + pzr_b[:, 0:_F] + lb[:, 0:_F])
    r = jax.nn.sigmoid(pr + pzr_b[:, _F:2 * _F] + lb[:, _F:2 * _F])
    ph_b = jnp.dot(hs * r, lbh_ref[...], precision=prec,
                   preferred_element_type=jnp.float32)
    htil = jnp.tanh(ph + ph_b + lb[:, 2 * _F:3 * _F])
    hn = z * hs + (1.0 - z) * htil
    h_ref[...] = hn
    y = jnp.maximum(hn, 0.0)
    y_ref[...] = y
    if with_head:
        out_ref[...] = jnp.dot(y, linw_ref[...], precision=prec,
                               preferred_element_type=jnp.float32) + linb_ref[...]
    else:
        out_ref[...] = jnp.zeros(out_ref.shape, out_ref.dtype)


def _cell_call(partsl, partsr, hs, degf, wcat, bcat, ltop, lbzr, lbh, lbias,
               linw, linb, with_head):
    grid = _N // _BN
    half_spec = pl.BlockSpec((_BN, _FH), lambda i: (i, 0))
    row_spec = pl.BlockSpec((_BN, _F), lambda i: (i, 0))
    deg_spec = pl.BlockSpec((_BN, 1), lambda i: (i, 0))
    full = lambda *shape: pl.BlockSpec(shape, lambda i: tuple(0 for _ in shape))
    out_shapes = [
        jax.ShapeDtypeStruct((_N, _F), jnp.float32),
        jax.ShapeDtypeStruct((_N, _F), jnp.float32),
        jax.ShapeDtypeStruct((_N, 1), jnp.float32),
    ]
    out_specs = [row_spec, row_spec, pl.BlockSpec((_BN, 1), lambda i: (i, 0))]
    h, y, out = pl.pallas_call(
        functools.partial(_cell_body, with_head),
        grid=(grid,),
        in_specs=[half_spec, half_spec, half_spec, half_spec, row_spec,
                  deg_spec,
                  full(_F, 3 * _F), full(1, 3 * _F), full(_F, 3 * _F),
                  full(_F, 2 * _F), full(_F, _F), full(1, 3 * _F),
                  full(_F, 1), full(1, 1)],
        out_specs=out_specs,
        out_shape=out_shapes,
    )(partsl[0], partsl[1], partsr[0], partsr[1], hs, degf, wcat, bcat, ltop,
      lbzr, lbh, lbias, linw, linb)
    return h, y, out


def _edge_layout(idx_or_w, pad_value):
    a = idx_or_w.reshape(_NW, _PER_W)
    a = jnp.pad(a, ((0, 0), (0, _PER_W_PAD - _PER_W)), constant_values=pad_value)
    return a.reshape(_NW, _NCHUNK, _CHUNK)


def _pack_cell_weights(Wz, bz, Lzw, Lzb, Wr, br, Lrw, Lrb, Wh, bh, Lhw, Lhb):
    wcat = jnp.concatenate([Wz, Wr, Wh], axis=1)
    bcat = jnp.concatenate([bz, br, bh]).reshape(1, 3 * _F)
    ltop = jnp.concatenate([Lzw[:_F], Lrw[:_F], Lhw[:_F]], axis=1)
    lbzr = jnp.concatenate([Lzw[_F:], Lrw[_F:]], axis=1)
    lbh = Lhw[_F:]
    lbias = jnp.concatenate([Lzb, Lrb, Lhb]).reshape(1, 3 * _F)
    return wcat, bcat, ltop, lbzr, lbh, lbias


def kernel(x, edge0, edge1, edge_weight0, edge_weight1, prev_h0, prev_h1,
           deg0, deg1, deg2, time, status,
           Wz1, bz1, Lz1w, Lz1b, Wr1, br1, Lr1w, Lr1b, Wh1, bh1, Lh1w, Lh1b,
           Wz2, bz2, Lz2w, Lz2b, Wr2, br2, Lr2w, Lr2b, Wh2, bh2, Lh2w, Lh2b,
           lin_w, lin_b):
    del deg0, time, status
    src1 = _edge_layout(edge1[0].astype(jnp.int32), 0)
    dst1 = _edge_layout(edge1[1].astype(jnp.int32), 0)
    ew1 = _edge_layout(edge_weight1, 0.0)
    src0 = _edge_layout(edge0[0].astype(jnp.int32), 0)
    dst0 = _edge_layout(edge0[1].astype(jnp.int32), 0)
    ew0 = _edge_layout(edge_weight0, 0.0)

    w1 = _pack_cell_weights(Wz1, bz1, Lz1w, Lz1b, Wr1, br1, Lr1w, Lr1b,
                            Wh1, bh1, Lh1w, Lh1b)
    w2 = _pack_cell_weights(Wz2, bz2, Lz2w, Lz2b, Wr2, br2, Lr2w, Lr2b,
                            Wh2, bh2, Lh2w, Lh2b)
    deg2f = deg2.astype(jnp.float32).reshape(_N, 1)
    deg1f = deg1.astype(jnp.float32).reshape(_N, 1)
    linw = lin_w.reshape(_F, 1)
    linb = lin_b.reshape(1, 1)

    xl = x[:, :_FH]
    xr = x[:, _FH:]
    p1l, p1r = _sc_segsum(xl, xr, src1, dst1, ew1)
    h1, y1, _ = _cell_call(p1l, p1r, prev_h0, deg2f, *w1, linw, linb,
                           with_head=False)
    p2l, p2r = _sc_segsum(y1[:, :_FH], y1[:, _FH:], src0, dst0, ew0)
    h2, _, out = _cell_call(p2l, p2r, prev_h1, deg1f, *w2, linw, linb,
                            with_head=True)
    return (out, h1, h2)
